# baseline (speedup 1.0000x reference)
import functools

import jax
import jax.numpy as jnp
from jax import lax
from jax.experimental import pallas as pl
from jax.experimental.pallas import tpu as pltpu
from jax.experimental.pallas import tpu_sc as plsc

_NUM_CORES = 2
_NUM_SUBCORES = 16
_NUM_WORKERS = _NUM_CORES * _NUM_SUBCORES


def _fused_table_body(tok_ref, w_ref, b_ref, out_ref):
    out_ref[...] = (
        jnp.dot(tok_ref[...], w_ref[...], preferred_element_type=jnp.float32)
        + b_ref[...]
    )


def _make_fused_table(V, D):
    return pl.pallas_call(
        _fused_table_body,
        out_shape=jax.ShapeDtypeStruct((V, D), jnp.float32),
    )


def _make_gather(V, D, B):
    assert B % _NUM_WORKERS == 0
    b_per_w = B // _NUM_WORKERS
    chunk = 64
    assert b_per_w % chunk == 0
    n_chunks = b_per_w // chunk

    mesh = plsc.VectorSubcoreMesh(core_axis_name="c", subcore_axis_name="s")

    @functools.partial(
        pl.kernel,
        mesh=mesh,
        compiler_params=pltpu.CompilerParams(use_tc_tiling_on_sc=False),
        out_type=jax.ShapeDtypeStruct((B, D), jnp.float32),
        scratch_types=[
            pltpu.VMEM((b_per_w,), jnp.int32),
            pltpu.VMEM((chunk, D), jnp.float32),
            pltpu.SemaphoreType.DMA,
        ],
    )
    def gather_kernel(table_hbm, idx_hbm, out_hbm, idx_v, rows_v, sem):
        wid = lax.axis_index("s") * _NUM_CORES + lax.axis_index("c")
        base = wid * b_per_w
        pltpu.sync_copy(idx_hbm.at[pl.ds(base, b_per_w)], idx_v)

        def body(c, _):
            off = c * chunk
            pltpu.async_copy(
                table_hbm.at[idx_v.at[pl.ds(off, chunk)]], rows_v, sem
            ).wait()
            pltpu.sync_copy(rows_v, out_hbm.at[pl.ds(base + off, chunk)])
            return 0

        lax.fori_loop(0, n_chunks, body, 0)

    return gather_kernel


def kernel(idx, tok_table, pos_table, W, b):
    del pos_table
    V, E = tok_table.shape
    D = W.shape[1]
    Bdim, T = idx.shape
    B = Bdim * T

    table = _make_fused_table(V, D)(tok_table, W, b.reshape(1, D))
    flat_idx = idx.reshape(B).astype(jnp.int32)
    out = _make_gather(V, D, B)(table, flat_idx)
    return out.reshape(Bdim, T, D)


# speedup vs baseline: 1.0123x; 1.0123x over previous
import functools

import jax
import jax.numpy as jnp
from jax import lax
from jax.experimental import pallas as pl
from jax.experimental.pallas import tpu as pltpu
from jax.experimental.pallas import tpu_sc as plsc

_NUM_CORES = 2
_NUM_SUBCORES = 16
_NUM_WORKERS = _NUM_CORES * _NUM_SUBCORES


def _fused_table_body(tok_ref, w_ref, b_ref, out_ref):
    out_ref[...] = (
        jnp.dot(tok_ref[...], w_ref[...], preferred_element_type=jnp.float32)
        + b_ref[...]
    )


def _make_fused_table(V, D):
    return pl.pallas_call(
        _fused_table_body,
        out_shape=jax.ShapeDtypeStruct((V, D), jnp.float32),
    )


def _make_gather(V, D, B):
    assert B % _NUM_WORKERS == 0
    b_per_w = B // _NUM_WORKERS
    chunk = 64
    assert b_per_w % chunk == 0
    n_chunks = b_per_w // chunk

    mesh = plsc.VectorSubcoreMesh(core_axis_name="c", subcore_axis_name="s")

    @functools.partial(
        pl.kernel,
        mesh=mesh,
        compiler_params=pltpu.CompilerParams(use_tc_tiling_on_sc=False),
        out_type=jax.ShapeDtypeStruct((B, D), jnp.float32),
        scratch_types=[
            pltpu.VMEM((b_per_w,), jnp.int32),
            pltpu.VMEM((2, chunk, D), jnp.float32),
            pltpu.SemaphoreType.DMA,
            pltpu.SemaphoreType.DMA,
            pltpu.SemaphoreType.DMA,
            pltpu.SemaphoreType.DMA,
        ],
    )
    def gather_kernel(table_hbm, idx_hbm, out_hbm, idx_v, rows_v, ga, gb, sa, sb):
        wid = lax.axis_index("s") * _NUM_CORES + lax.axis_index("c")
        base = wid * b_per_w
        pltpu.sync_copy(idx_hbm.at[pl.ds(base, b_per_w)], idx_v)

        def start_gather(c, buf, sem):
            return pltpu.async_copy(
                table_hbm.at[idx_v.at[pl.ds(c * chunk, chunk)]],
                rows_v.at[buf],
                sem,
            )

        def start_scatter(c, buf, sem):
            return pltpu.async_copy(
                rows_v.at[buf], out_hbm.at[pl.ds(base + c * chunk, chunk)], sem
            )

        def pair(p, _):
            c = 2 * p
            cp0 = start_gather(c, 0, ga)
            cp1 = start_gather(c + 1, 1, gb)
            cp0.wait()
            s0 = start_scatter(c, 0, sa)
            cp1.wait()
            s1 = start_scatter(c + 1, 1, sb)
            s0.wait()
            s1.wait()
            return 0

        lax.fori_loop(0, n_chunks // 2, pair, 0)

    return gather_kernel


def kernel(idx, tok_table, pos_table, W, b):
    del pos_table
    V, E = tok_table.shape
    D = W.shape[1]
    Bdim, T = idx.shape
    B = Bdim * T

    table = _make_fused_table(V, D)(tok_table, W, b.reshape(1, D))
    flat_idx = idx.reshape(B).astype(jnp.int32)
    out = _make_gather(V, D, B)(table, flat_idx)
    return out.reshape(Bdim, T, D)


# speedup vs baseline: 2.0021x; 1.9778x over previous
import functools

import jax
import jax.numpy as jnp
from jax import lax
from jax.experimental import pallas as pl
from jax.experimental.pallas import tpu as pltpu
from jax.experimental.pallas import tpu_sc as plsc

_NUM_CORES = 2
_NUM_SUBCORES = 16
_NUM_WORKERS = _NUM_CORES * _NUM_SUBCORES


def _make_sc_gather(V, E, B):
    assert B % _NUM_WORKERS == 0
    b_per_w = B // _NUM_WORKERS
    chunk = 128
    assert b_per_w % chunk == 0
    n_chunks = b_per_w // chunk

    mesh = plsc.VectorSubcoreMesh(core_axis_name="c", subcore_axis_name="s")

    @functools.partial(
        pl.kernel,
        mesh=mesh,
        compiler_params=pltpu.CompilerParams(use_tc_tiling_on_sc=False),
        out_type=jax.ShapeDtypeStruct((B, E), jnp.float32),
        scratch_types=[
            pltpu.VMEM((b_per_w,), jnp.int32),
            pltpu.VMEM((2, chunk, E), jnp.float32),
            pltpu.SemaphoreType.DMA,
            pltpu.SemaphoreType.DMA,
            pltpu.SemaphoreType.DMA,
            pltpu.SemaphoreType.DMA,
        ],
    )
    def gather_kernel(table_hbm, idx_hbm, out_hbm, idx_v, rows_v, ga, gb, sa, sb):
        wid = lax.axis_index("s") * _NUM_CORES + lax.axis_index("c")
        base = wid * b_per_w
        pltpu.sync_copy(idx_hbm.at[pl.ds(base, b_per_w)], idx_v)

        def start_gather(c, buf, sem):
            return pltpu.async_copy(
                table_hbm.at[idx_v.at[pl.ds(c * chunk, chunk)]],
                rows_v.at[buf],
                sem,
            )

        def start_scatter(c, buf, sem):
            return pltpu.async_copy(
                rows_v.at[buf], out_hbm.at[pl.ds(base + c * chunk, chunk)], sem
            )

        def pair(p, _):
            c = 2 * p
            g0 = start_gather(c, 0, ga)
            g1 = start_gather(c + 1, 1, gb)
            g0.wait()
            s0 = start_scatter(c, 0, sa)
            g1.wait()
            s1 = start_scatter(c + 1, 1, sb)
            s0.wait()
            s1.wait()
            return 0

        lax.fori_loop(0, n_chunks // 2, pair, 0)

    return gather_kernel


def _head_body(emb_ref, w_ref, b_ref, out_ref):
    emb = emb_ref[...].astype(jnp.bfloat16)
    w = w_ref[...].astype(jnp.bfloat16)
    out_ref[...] = (
        jnp.dot(emb, w, preferred_element_type=jnp.float32) + b_ref[...]
    )


def _make_head(B, E, D, bm):
    assert B % bm == 0
    return pl.pallas_call(
        _head_body,
        grid=(B // bm,),
        in_specs=[
            pl.BlockSpec((bm, E), lambda i: (i, 0)),
            pl.BlockSpec((E, D), lambda i: (0, 0)),
            pl.BlockSpec((1, D), lambda i: (0, 0)),
        ],
        out_specs=pl.BlockSpec((bm, D), lambda i: (i, 0)),
        out_shape=jax.ShapeDtypeStruct((B, D), jnp.float32),
    )


def kernel(idx, tok_table, pos_table, W, b):
    del pos_table
    V, E = tok_table.shape
    D = W.shape[1]
    Bdim, T = idx.shape
    B = Bdim * T

    flat_idx = idx.reshape(B).astype(jnp.int32)
    emb = _make_sc_gather(V, E, B)(tok_table, flat_idx)
    out = _make_head(B, E, D, bm=2048)(emb, W, b.reshape(1, D))
    return out.reshape(Bdim, T, D)
